# Initial kernel scaffold; baseline (speedup 1.0000x reference)
import jax
import jax.numpy as jnp
from jax.experimental import pallas as pl


def kernel(x, edge_index, W1, bn_gamma, bn_beta, W2, b2, Wx1, bx1, Wx2, Wm, bm):
    raise NotImplementedError("write your pallas kernel here")


# speedup vs baseline: 5.7121x; 5.7121x over previous
import functools

import jax
import jax.numpy as jnp
from jax import lax
from jax.experimental import pallas as pl
from jax.experimental.pallas import tpu as pltpu
from jax.experimental.pallas import tpu_sc as plsc

BN_EPS = 1e-5
C_WEIGHT = 1.0

NC = 2
NS = 16
L = 16
NW = NC * NS

G_DMA = 80
CH = 2000


def _psi(v):
    return jnp.sign(v) * jnp.log(jnp.abs(v) + 1.0)


def _iota16():
    return lax.iota(jnp.int32, 16)


def _make_sc_edge_features(n_nodes, n_edges):
    epw = n_edges // NW
    nchunk = epw // CH
    ng_dma = CH // G_DMA
    ngc = CH // L
    mesh = plsc.VectorSubcoreMesh(core_axis_name="c", subcore_axis_name="s")
    fdt = jnp.float32

    @functools.partial(
        pl.kernel,
        out_type=tuple(jax.ShapeDtypeStruct((n_edges,), fdt) for _ in range(5)),
        mesh=mesh,
        scratch_types=[
            pltpu.VMEM((1, ng_dma, G_DMA), jnp.int32),
            pltpu.VMEM((1, ng_dma, G_DMA), jnp.int32),
            pltpu.VMEM((CH, 4), fdt),
            pltpu.VMEM((CH, 4), fdt),
            pltpu.VMEM((CH,), fdt),
            pltpu.VMEM((CH,), fdt),
            pltpu.VMEM((CH,), fdt),
            pltpu.VMEM((CH,), fdt),
            pltpu.VMEM((CH,), fdt),
            pltpu.SemaphoreType.DMA,
        ],
        compiler_params=pltpu.CompilerParams(needs_layout_passes=False, use_tc_tiling_on_sc=False),
    )
    def edge_features(xpad, idx_i3, idx_j3,
                      nrm_out, dot_out, d0_out, d1_out, d2_out,
                      idx_i_v, idx_j_v, xi_v, xj_v,
                      nrm_v, dot_v, d0_v, d1_v, d2_v, sem):
        cid = lax.axis_index("c")
        sid = lax.axis_index("s")
        wid = sid * NC + cid

        c0 = jnp.full((16,), 0, jnp.int32)
        c1 = jnp.full((16,), 1, jnp.int32)
        c2 = jnp.full((16,), 2, jnp.int32)

        def chunk_body(c, _):
            gc = wid * nchunk + c
            pltpu.sync_copy(idx_i3.at[pl.ds(gc, 1)], idx_i_v)
            pltpu.sync_copy(idx_j3.at[pl.ds(gc, 1)], idx_j_v)

            def gather_body(g, _):
                ci = pltpu.async_copy(
                    xpad.at[idx_i_v.at[0, g]], xi_v.at[pl.ds(g * G_DMA, G_DMA)], sem)
                cj = pltpu.async_copy(
                    xpad.at[idx_j_v.at[0, g]], xj_v.at[pl.ds(g * G_DMA, G_DMA)], sem)
                ci.wait()
                cj.wait()
                return 0

            lax.fori_loop(0, ng_dma, gather_body, 0)

            def comp_body(gg, _):
                e0 = gg * L
                evec = e0 + _iota16()
                xi0 = plsc.load_gather(xi_v, [evec, c0])
                xi1 = plsc.load_gather(xi_v, [evec, c1])
                xi2 = plsc.load_gather(xi_v, [evec, c2])
                xj0 = plsc.load_gather(xj_v, [evec, c0])
                xj1 = plsc.load_gather(xj_v, [evec, c1])
                xj2 = plsc.load_gather(xj_v, [evec, c2])
                d0 = xi0 - xj0
                d1 = xi1 - xj1
                d2 = xi2 - xj2
                nrm_v[pl.ds(e0, L)] = d0 * d0 + d1 * d1 + d2 * d2
                dot_v[pl.ds(e0, L)] = xi0 * xj0 + xi1 * xj1 + xi2 * xj2
                d0_v[pl.ds(e0, L)] = d0
                d1_v[pl.ds(e0, L)] = d1
                d2_v[pl.ds(e0, L)] = d2
                return 0

            lax.fori_loop(0, ngc, comp_body, 0)

            ebase = wid * epw + c * CH
            pltpu.sync_copy(nrm_v, nrm_out.at[pl.ds(ebase, CH)])
            pltpu.sync_copy(dot_v, dot_out.at[pl.ds(ebase, CH)])
            pltpu.sync_copy(d0_v, d0_out.at[pl.ds(ebase, CH)])
            pltpu.sync_copy(d1_v, d1_out.at[pl.ds(ebase, CH)])
            pltpu.sync_copy(d2_v, d2_out.at[pl.ds(ebase, CH)])
            return 0

        lax.fori_loop(0, nchunk, chunk_body, 0)

    return edge_features


def _make_sc_scatter(n_pad, n_edges):
    epw = n_edges // NW
    nchunk = epw // CH
    ng_dma = CH // G_DMA
    ngc = CH // L
    stripe = n_pad // NS
    mesh = plsc.VectorSubcoreMesh(core_axis_name="c", subcore_axis_name="s")
    fdt = jnp.float32

    @functools.partial(
        pl.kernel,
        out_type=jax.ShapeDtypeStruct((NC, n_pad, 4), fdt),
        mesh=mesh,
        scratch_types=[
            pltpu.VMEM((1, ng_dma, G_DMA), jnp.int32),
            pltpu.VMEM((CH,), fdt),
            pltpu.VMEM((CH,), fdt),
            pltpu.VMEM((CH,), fdt),
            pltpu.VMEM((CH,), fdt),
            pltpu.VMEM((CH, 4), fdt),
            pltpu.VMEM_SHARED((n_pad, 4), fdt),
            pltpu.SemaphoreType.DMA,
        ],
        compiler_params=pltpu.CompilerParams(needs_layout_passes=False, use_tc_tiling_on_sc=False),
    )
    def scatter_mean(idx_i3, d0_in, d1_in, d2_in, phi_in, zeros_init,
                     acc_out,
                     idx_v, d0_v, d1_v, d2_v, phi_v, rows_v, acc_sh, sem):
        cid = lax.axis_index("c")
        sid = lax.axis_index("s")
        wid = sid * NC + cid

        c0 = jnp.full((16,), 0, jnp.int32)
        c1 = jnp.full((16,), 1, jnp.int32)
        c2 = jnp.full((16,), 2, jnp.int32)
        c3 = jnp.full((16,), 3, jnp.int32)
        ones = jnp.full((16,), 1.0, fdt)

        @pl.when(sid == 0)
        def _():
            pltpu.sync_copy(zeros_init, acc_sh)

        def fill_body(gg, _):
            evec = gg * L + _iota16()
            plsc.store_scatter(rows_v, [evec, c3], ones)
            return 0

        lax.fori_loop(0, ngc, fill_body, 0)
        plsc.subcore_barrier()

        def chunk_body(c, _):
            gc = wid * nchunk + c
            ebase = wid * epw + c * CH
            pltpu.sync_copy(idx_i3.at[pl.ds(gc, 1)], idx_v)
            pltpu.sync_copy(d0_in.at[pl.ds(ebase, CH)], d0_v)
            pltpu.sync_copy(d1_in.at[pl.ds(ebase, CH)], d1_v)
            pltpu.sync_copy(d2_in.at[pl.ds(ebase, CH)], d2_v)
            pltpu.sync_copy(phi_in.at[pl.ds(ebase, CH)], phi_v)

            def comp_body(gg, _):
                e0 = gg * L
                evec = e0 + _iota16()
                ph = phi_v[pl.ds(e0, L)]
                u0 = jnp.clip(d0_v[pl.ds(e0, L)] * ph, -100.0, 100.0)
                u1 = jnp.clip(d1_v[pl.ds(e0, L)] * ph, -100.0, 100.0)
                u2 = jnp.clip(d2_v[pl.ds(e0, L)] * ph, -100.0, 100.0)
                plsc.store_scatter(rows_v, [evec, c0], u0)
                plsc.store_scatter(rows_v, [evec, c1], u1)
                plsc.store_scatter(rows_v, [evec, c2], u2)
                return 0

            lax.fori_loop(0, ngc, comp_body, 0)

            def scat_body(g, _):
                pltpu.sync_copy(rows_v.at[pl.ds(g * G_DMA, G_DMA)],
                                acc_sh.at[idx_v.at[0, g]], add=True)
                return 0

            lax.fori_loop(0, ng_dma, scat_body, 0)
            return 0

        lax.fori_loop(0, nchunk, chunk_body, 0)
        plsc.subcore_barrier()

        r0 = sid * stripe
        pltpu.sync_copy(acc_sh.at[pl.ds(r0, stripe)],
                        acc_out.at[cid, pl.ds(r0, stripe)])

    return scatter_mean


def _stats_kernel(n_ref, d_ref, o_ref, acc_ref):
    b = pl.program_id(0)
    nblk = pl.num_programs(0)
    pn = _psi(n_ref[0, 0, :].reshape(-1, 128))
    pd = _psi(d_ref[0, 0, :].reshape(-1, 128))

    @pl.when(b == 0)
    def _():
        acc_ref[...] = jnp.zeros_like(acc_ref)

    acc_ref[0, :] += jnp.sum(pn, axis=0)
    acc_ref[1, :] += jnp.sum(pd, axis=0)
    acc_ref[2, :] += jnp.sum(pn * pn, axis=0)
    acc_ref[3, :] += jnp.sum(pd * pd, axis=0)
    acc_ref[4, :] += jnp.sum(pn * pd, axis=0)

    @pl.when(b == nblk - 1)
    def _():
        o_ref[...] = jnp.sum(acc_ref[...], axis=1, keepdims=True)


def _mlp_kernel(n_ref, d_ref, w0_ref, w1_ref, b1_ref, W2T_ref, b2_ref,
                wm_ref, bm_ref, Wx1T_ref, bx1_ref, wx2_ref, m_ref, phi_ref):
    pn = _psi(n_ref[0, 0, :])[:, None]
    pd = _psi(d_ref[0, 0, :])[:, None]
    h = pn * w0_ref[...] + pd * w1_ref[...] + b1_ref[...]
    h = jnp.maximum(h, 0.0)
    m = jnp.dot(h, W2T_ref[...], preferred_element_type=jnp.float32)
    m = jnp.maximum(m + b2_ref[...], 0.0)
    w = jax.nn.sigmoid(
        jnp.sum(m * wm_ref[...], axis=1, keepdims=True) + bm_ref[...])
    mg = m * w
    m_ref[...] = mg
    t = jnp.dot(mg, Wx1T_ref[...], preferred_element_type=jnp.float32)
    t = jnp.maximum(t + bx1_ref[...], 0.0)
    phi_ref[0, 0, :] = jnp.sum(t * wx2_ref[...], axis=1)


def _combine_kernel(x_ref, acc_ref, o_ref):
    a = acc_ref[0] + acc_ref[1]
    cnt = a[:, 3:4]
    o_ref[...] = x_ref[...] + (a / jnp.maximum(cnt, 1.0)) * C_WEIGHT


def kernel(x, edge_index, W1, bn_gamma, bn_beta, W2, b2, Wx1, bx1, Wx2, Wm, bm):
    n_nodes = x.shape[0]
    n_edges = edge_index.shape[1]
    H = W1.shape[0]
    fdt = jnp.float32

    i_idx = edge_index[0].astype(jnp.int32)
    j_idx = edge_index[1].astype(jnp.int32)
    nchunks_total = n_edges // CH
    idx_i3 = i_idx.reshape(nchunks_total, CH // G_DMA, G_DMA)
    idx_j3 = j_idx.reshape(nchunks_total, CH // G_DMA, G_DMA)
    xpad = jnp.concatenate([x, jnp.zeros((n_nodes, 1), fdt)], axis=1)

    nrm, dot, d0, d1, d2 = _make_sc_edge_features(n_nodes, n_edges)(
        xpad, idx_i3, idx_j3)

    be2 = 32000
    g2 = n_edges // be2
    sums = pl.pallas_call(
        _stats_kernel,
        grid=(g2,),
        in_specs=[
            pl.BlockSpec((1, 1, be2), lambda b: (b, 0, 0)),
            pl.BlockSpec((1, 1, be2), lambda b: (b, 0, 0)),
        ],
        out_specs=pl.BlockSpec((8, 1), lambda b: (0, 0)),
        out_shape=jax.ShapeDtypeStruct((8, 1), fdt),
        scratch_shapes=[pltpu.VMEM((8, 128), fdt)],
    )(nrm.reshape(g2, 1, be2), dot.reshape(g2, 1, be2))

    s = sums[:, 0]
    inv_e = 1.0 / n_edges
    mean_n = s[0] * inv_e
    mean_d = s[1] * inv_e
    var_n = s[2] * inv_e - mean_n * mean_n
    var_d = s[3] * inv_e - mean_d * mean_d
    cov_nd = s[4] * inv_e - mean_n * mean_d
    w1n = W1[:, 0]
    w1d = W1[:, 1]
    mu = w1n * mean_n + w1d * mean_d
    var = w1n * w1n * var_n + 2.0 * w1n * w1d * cov_nd + w1d * w1d * var_d
    scale = bn_gamma / jnp.sqrt(var + BN_EPS)
    weff0 = (w1n * scale)[None, :]
    weff1 = (w1d * scale)[None, :]
    beff = (bn_beta - mu * scale)[None, :]

    be3 = 8000
    g3 = n_edges // be3
    full = lambda shp: pl.BlockSpec(shp, lambda b: tuple(0 for _ in shp))
    m_ij, phi3 = pl.pallas_call(
        _mlp_kernel,
        grid=(g3,),
        in_specs=[
            pl.BlockSpec((1, 1, be3), lambda b: (b, 0, 0)),
            pl.BlockSpec((1, 1, be3), lambda b: (b, 0, 0)),
            full((1, H)), full((1, H)), full((1, H)),
            full((H, H)), full((1, H)),
            full((1, H)), full((1, 1)),
            full((H, H)), full((1, H)), full((1, H)),
        ],
        out_specs=[
            pl.BlockSpec((be3, H), lambda b: (b, 0)),
            pl.BlockSpec((1, 1, be3), lambda b: (b, 0, 0)),
        ],
        out_shape=[
            jax.ShapeDtypeStruct((n_edges, H), fdt),
            jax.ShapeDtypeStruct((g3, 1, be3), fdt),
        ],
    )(nrm.reshape(g3, 1, be3), dot.reshape(g3, 1, be3),
      weff0, weff1, beff, W2.T, b2[None, :], Wm, bm[None, :],
      Wx1.T, bx1[None, :], Wx2)
    phi = phi3.reshape(n_edges)

    n_pad = ((n_nodes + NS * 8 - 1) // (NS * 8)) * NS * 8
    zeros_init = jnp.zeros((n_pad, 4), fdt)
    acc = _make_sc_scatter(n_pad, n_edges)(
        idx_i3, d0, d1, d2, phi, zeros_init)

    x4 = jnp.pad(x, ((0, n_pad - n_nodes), (0, 1)))
    br = n_pad // 8
    x4t = pl.pallas_call(
        _combine_kernel,
        grid=(8,),
        in_specs=[
            pl.BlockSpec((br, 4), lambda b: (b, 0)),
            pl.BlockSpec((NC, br, 4), lambda b: (0, b, 0)),
        ],
        out_specs=pl.BlockSpec((br, 4), lambda b: (b, 0)),
        out_shape=jax.ShapeDtypeStruct((n_pad, 4), fdt),
    )(x4, acc)
    x_tilde = x4t[:n_nodes, :3]

    return (x_tilde, m_ij)


# speedup vs baseline: 8.6306x; 1.5109x over previous
import functools

import jax
import jax.numpy as jnp
from jax import lax
from jax.experimental import pallas as pl
from jax.experimental.pallas import tpu as pltpu
from jax.experimental.pallas import tpu_sc as plsc

BN_EPS = 1e-5
C_WEIGHT = 1.0

NC = 2
NS = 16
L = 16
NW = NC * NS

G_DMA = 80
CH = 2000


def _psi(v):
    return jnp.sign(v) * jnp.log(jnp.abs(v) + 1.0)


def _iota16():
    return lax.iota(jnp.int32, 16)


def _make_sc_edge_features(n_nodes, n_edges):
    epw = n_edges // NW
    nchunk = epw // CH
    ng_dma = CH // G_DMA
    ngc = CH // L
    mesh = plsc.VectorSubcoreMesh(core_axis_name="c", subcore_axis_name="s")
    fdt = jnp.float32

    @functools.partial(
        pl.kernel,
        out_type=tuple(jax.ShapeDtypeStruct((n_edges,), fdt) for _ in range(5)),
        mesh=mesh,
        scratch_types=[
            pltpu.VMEM((1, ng_dma, G_DMA), jnp.int32),
            pltpu.VMEM((1, ng_dma, G_DMA), jnp.int32),
            pltpu.VMEM((CH, 4), fdt),
            pltpu.VMEM((CH, 4), fdt),
            pltpu.VMEM((CH,), fdt),
            pltpu.VMEM((CH,), fdt),
            pltpu.VMEM((CH,), fdt),
            pltpu.VMEM((CH,), fdt),
            pltpu.VMEM((CH,), fdt),
            pltpu.SemaphoreType.DMA,
        ],
        compiler_params=pltpu.CompilerParams(needs_layout_passes=False, use_tc_tiling_on_sc=False),
    )
    def edge_features(xpad, idx_i3, idx_j3,
                      nrm_out, dot_out, d0_out, d1_out, d2_out,
                      idx_i_v, idx_j_v, xi_v, xj_v,
                      nrm_v, dot_v, d0_v, d1_v, d2_v, sem):
        cid = lax.axis_index("c")
        sid = lax.axis_index("s")
        wid = sid * NC + cid

        c0 = jnp.full((16,), 0, jnp.int32)
        c1 = jnp.full((16,), 1, jnp.int32)
        c2 = jnp.full((16,), 2, jnp.int32)

        def chunk_body(c, _):
            gc = wid * nchunk + c
            pltpu.sync_copy(idx_i3.at[pl.ds(gc, 1)], idx_i_v)
            pltpu.sync_copy(idx_j3.at[pl.ds(gc, 1)], idx_j_v)

            def gather_issue(g, _):
                pltpu.async_copy(
                    xpad.at[idx_i_v.at[0, g]], xi_v.at[pl.ds(g * G_DMA, G_DMA)], sem)
                pltpu.async_copy(
                    xpad.at[idx_j_v.at[0, g]], xj_v.at[pl.ds(g * G_DMA, G_DMA)], sem)
                return 0

            def gather_drain(g, _):
                pltpu.make_async_copy(
                    xpad.at[idx_i_v.at[0, g]], xi_v.at[pl.ds(g * G_DMA, G_DMA)], sem).wait()
                pltpu.make_async_copy(
                    xpad.at[idx_j_v.at[0, g]], xj_v.at[pl.ds(g * G_DMA, G_DMA)], sem).wait()
                return 0

            lax.fori_loop(0, ng_dma, gather_issue, 0)
            lax.fori_loop(0, ng_dma, gather_drain, 0)

            def comp_body(gg, _):
                e0 = gg * L
                evec = e0 + _iota16()
                xi0 = plsc.load_gather(xi_v, [evec, c0])
                xi1 = plsc.load_gather(xi_v, [evec, c1])
                xi2 = plsc.load_gather(xi_v, [evec, c2])
                xj0 = plsc.load_gather(xj_v, [evec, c0])
                xj1 = plsc.load_gather(xj_v, [evec, c1])
                xj2 = plsc.load_gather(xj_v, [evec, c2])
                d0 = xi0 - xj0
                d1 = xi1 - xj1
                d2 = xi2 - xj2
                nrm_v[pl.ds(e0, L)] = d0 * d0 + d1 * d1 + d2 * d2
                dot_v[pl.ds(e0, L)] = xi0 * xj0 + xi1 * xj1 + xi2 * xj2
                d0_v[pl.ds(e0, L)] = d0
                d1_v[pl.ds(e0, L)] = d1
                d2_v[pl.ds(e0, L)] = d2
                return 0

            lax.fori_loop(0, ngc, comp_body, 0)

            ebase = wid * epw + c * CH
            pltpu.sync_copy(nrm_v, nrm_out.at[pl.ds(ebase, CH)])
            pltpu.sync_copy(dot_v, dot_out.at[pl.ds(ebase, CH)])
            pltpu.sync_copy(d0_v, d0_out.at[pl.ds(ebase, CH)])
            pltpu.sync_copy(d1_v, d1_out.at[pl.ds(ebase, CH)])
            pltpu.sync_copy(d2_v, d2_out.at[pl.ds(ebase, CH)])
            return 0

        lax.fori_loop(0, nchunk, chunk_body, 0)

    return edge_features


def _make_sc_scatter(n_pad, n_edges):
    epw = n_edges // NW
    nchunk = epw // CH
    ng_dma = CH // G_DMA
    ngc = CH // L
    stripe = n_pad // NS
    mesh = plsc.VectorSubcoreMesh(core_axis_name="c", subcore_axis_name="s")
    fdt = jnp.float32

    @functools.partial(
        pl.kernel,
        out_type=jax.ShapeDtypeStruct((NC, n_pad, 4), fdt),
        mesh=mesh,
        scratch_types=[
            pltpu.VMEM((1, ng_dma, G_DMA), jnp.int32),
            pltpu.VMEM((CH,), fdt),
            pltpu.VMEM((CH,), fdt),
            pltpu.VMEM((CH,), fdt),
            pltpu.VMEM((CH,), fdt),
            pltpu.VMEM((CH, 4), fdt),
            pltpu.VMEM_SHARED((n_pad, 4), fdt),
            pltpu.SemaphoreType.DMA,
        ],
        compiler_params=pltpu.CompilerParams(needs_layout_passes=False, use_tc_tiling_on_sc=False),
    )
    def scatter_mean(idx_i3, d0_in, d1_in, d2_in, phi_in, zeros_init,
                     acc_out,
                     idx_v, d0_v, d1_v, d2_v, phi_v, rows_v, acc_sh, sem):
        cid = lax.axis_index("c")
        sid = lax.axis_index("s")
        wid = sid * NC + cid

        c0 = jnp.full((16,), 0, jnp.int32)
        c1 = jnp.full((16,), 1, jnp.int32)
        c2 = jnp.full((16,), 2, jnp.int32)
        c3 = jnp.full((16,), 3, jnp.int32)
        ones = jnp.full((16,), 1.0, fdt)

        @pl.when(sid == 0)
        def _():
            pltpu.sync_copy(zeros_init, acc_sh)

        def fill_body(gg, _):
            evec = gg * L + _iota16()
            plsc.store_scatter(rows_v, [evec, c3], ones)
            return 0

        lax.fori_loop(0, ngc, fill_body, 0)
        plsc.subcore_barrier()

        def chunk_body(c, _):
            gc = wid * nchunk + c
            ebase = wid * epw + c * CH
            pltpu.sync_copy(idx_i3.at[pl.ds(gc, 1)], idx_v)
            pltpu.sync_copy(d0_in.at[pl.ds(ebase, CH)], d0_v)
            pltpu.sync_copy(d1_in.at[pl.ds(ebase, CH)], d1_v)
            pltpu.sync_copy(d2_in.at[pl.ds(ebase, CH)], d2_v)
            pltpu.sync_copy(phi_in.at[pl.ds(ebase, CH)], phi_v)

            def comp_body(gg, _):
                e0 = gg * L
                evec = e0 + _iota16()
                ph = phi_v[pl.ds(e0, L)]
                u0 = jnp.clip(d0_v[pl.ds(e0, L)] * ph, -100.0, 100.0)
                u1 = jnp.clip(d1_v[pl.ds(e0, L)] * ph, -100.0, 100.0)
                u2 = jnp.clip(d2_v[pl.ds(e0, L)] * ph, -100.0, 100.0)
                plsc.store_scatter(rows_v, [evec, c0], u0)
                plsc.store_scatter(rows_v, [evec, c1], u1)
                plsc.store_scatter(rows_v, [evec, c2], u2)
                return 0

            lax.fori_loop(0, ngc, comp_body, 0)

            def scat_issue(g, _):
                pltpu.async_copy(rows_v.at[pl.ds(g * G_DMA, G_DMA)],
                                 acc_sh.at[idx_v.at[0, g]], sem, add=True)
                return 0

            def scat_drain(g, _):
                pltpu.make_async_copy(rows_v.at[pl.ds(g * G_DMA, G_DMA)],
                                      acc_sh.at[idx_v.at[0, g]], sem).wait()
                return 0

            lax.fori_loop(0, ng_dma, scat_issue, 0)
            lax.fori_loop(0, ng_dma, scat_drain, 0)
            return 0

        lax.fori_loop(0, nchunk, chunk_body, 0)
        plsc.subcore_barrier()

        r0 = sid * stripe
        pltpu.sync_copy(acc_sh.at[pl.ds(r0, stripe)],
                        acc_out.at[cid, pl.ds(r0, stripe)])

    return scatter_mean


def _stats_kernel(n_ref, d_ref, o_ref, acc_ref):
    b = pl.program_id(0)
    nblk = pl.num_programs(0)
    pn = _psi(n_ref[0, 0, :].reshape(-1, 128))
    pd = _psi(d_ref[0, 0, :].reshape(-1, 128))

    @pl.when(b == 0)
    def _():
        acc_ref[...] = jnp.zeros_like(acc_ref)

    acc_ref[0, :] += jnp.sum(pn, axis=0)
    acc_ref[1, :] += jnp.sum(pd, axis=0)
    acc_ref[2, :] += jnp.sum(pn * pn, axis=0)
    acc_ref[3, :] += jnp.sum(pd * pd, axis=0)
    acc_ref[4, :] += jnp.sum(pn * pd, axis=0)

    @pl.when(b == nblk - 1)
    def _():
        o_ref[...] = jnp.sum(acc_ref[...], axis=1, keepdims=True)


def _mlp_kernel(n_ref, d_ref, w0_ref, w1_ref, b1_ref, W2T_ref, b2_ref,
                wmbc_ref, bm_ref, Wx1T_ref, bx1_ref, wx2c_ref, m_ref, phi_ref):
    pn = _psi(n_ref[0, 0, :])[:, None]
    pd = _psi(d_ref[0, 0, :])[:, None]
    h = pn * w0_ref[...] + pd * w1_ref[...] + b1_ref[...]
    h = jnp.maximum(h, 0.0)
    m = jnp.dot(h, W2T_ref[...], preferred_element_type=jnp.float32)
    m = jnp.maximum(m + b2_ref[...], 0.0)
    gate = jnp.dot(m, wmbc_ref[...], preferred_element_type=jnp.float32)
    w = jax.nn.sigmoid(gate + bm_ref[...])
    mg = m * w
    m_ref[...] = mg
    t = jnp.dot(mg, Wx1T_ref[...], preferred_element_type=jnp.float32)
    t = jnp.maximum(t + bx1_ref[...], 0.0)
    phi_ref[...] = jnp.dot(t, wx2c_ref[...], preferred_element_type=jnp.float32)


def _combine_kernel(x_ref, acc_ref, o_ref):
    a = acc_ref[0] + acc_ref[1]
    cnt = a[:, 3:4]
    o_ref[...] = x_ref[...] + (a / jnp.maximum(cnt, 1.0)) * C_WEIGHT


def kernel(x, edge_index, W1, bn_gamma, bn_beta, W2, b2, Wx1, bx1, Wx2, Wm, bm):
    n_nodes = x.shape[0]
    n_edges = edge_index.shape[1]
    H = W1.shape[0]
    fdt = jnp.float32

    i_idx = edge_index[0].astype(jnp.int32)
    j_idx = edge_index[1].astype(jnp.int32)
    nchunks_total = n_edges // CH
    idx_i3 = i_idx.reshape(nchunks_total, CH // G_DMA, G_DMA)
    idx_j3 = j_idx.reshape(nchunks_total, CH // G_DMA, G_DMA)
    xpad = jnp.concatenate([x, jnp.zeros((n_nodes, 1), fdt)], axis=1)

    nrm, dot, d0, d1, d2 = _make_sc_edge_features(n_nodes, n_edges)(
        xpad, idx_i3, idx_j3)

    be2 = 32000
    g2 = n_edges // be2
    sums = pl.pallas_call(
        _stats_kernel,
        grid=(g2,),
        in_specs=[
            pl.BlockSpec((1, 1, be2), lambda b: (b, 0, 0)),
            pl.BlockSpec((1, 1, be2), lambda b: (b, 0, 0)),
        ],
        out_specs=pl.BlockSpec((8, 1), lambda b: (0, 0)),
        out_shape=jax.ShapeDtypeStruct((8, 1), fdt),
        scratch_shapes=[pltpu.VMEM((8, 128), fdt)],
    )(nrm.reshape(g2, 1, be2), dot.reshape(g2, 1, be2))

    s = sums[:, 0]
    inv_e = 1.0 / n_edges
    mean_n = s[0] * inv_e
    mean_d = s[1] * inv_e
    var_n = s[2] * inv_e - mean_n * mean_n
    var_d = s[3] * inv_e - mean_d * mean_d
    cov_nd = s[4] * inv_e - mean_n * mean_d
    w1n = W1[:, 0]
    w1d = W1[:, 1]
    mu = w1n * mean_n + w1d * mean_d
    var = w1n * w1n * var_n + 2.0 * w1n * w1d * cov_nd + w1d * w1d * var_d
    scale = bn_gamma / jnp.sqrt(var + BN_EPS)
    weff0 = (w1n * scale)[None, :]
    weff1 = (w1d * scale)[None, :]
    beff = (bn_beta - mu * scale)[None, :]

    be3 = 8000
    g3 = n_edges // be3
    full = lambda shp: pl.BlockSpec(shp, lambda b: tuple(0 for _ in shp))
    m_ij, phi3 = pl.pallas_call(
        _mlp_kernel,
        grid=(g3,),
        in_specs=[
            pl.BlockSpec((1, 1, be3), lambda b: (b, 0, 0)),
            pl.BlockSpec((1, 1, be3), lambda b: (b, 0, 0)),
            full((1, H)), full((1, H)), full((1, H)),
            full((H, H)), full((1, H)),
            full((H, H)), full((1, 1)),
            full((H, H)), full((1, H)), full((H, 1)),
        ],
        out_specs=[
            pl.BlockSpec((be3, H), lambda b: (b, 0)),
            pl.BlockSpec((be3, 1), lambda b: (b, 0)),
        ],
        out_shape=[
            jax.ShapeDtypeStruct((n_edges, H), fdt),
            jax.ShapeDtypeStruct((n_edges, 1), fdt),
        ],
    )(nrm.reshape(g3, 1, be3), dot.reshape(g3, 1, be3),
      weff0, weff1, beff, W2.T, b2[None, :],
      jnp.broadcast_to(Wm.reshape(H, 1), (H, H)), bm[None, :],
      Wx1.T, bx1[None, :], Wx2.reshape(H, 1))
    phi = phi3.reshape(n_edges)

    n_pad = ((n_nodes + NS * 8 - 1) // (NS * 8)) * NS * 8
    zeros_init = jnp.zeros((n_pad, 4), fdt)
    acc = _make_sc_scatter(n_pad, n_edges)(
        idx_i3, d0, d1, d2, phi, zeros_init)

    x4 = jnp.pad(x, ((0, n_pad - n_nodes), (0, 1)))
    br = n_pad // 8
    x4t = pl.pallas_call(
        _combine_kernel,
        grid=(8,),
        in_specs=[
            pl.BlockSpec((br, 4), lambda b: (b, 0)),
            pl.BlockSpec((NC, br, 4), lambda b: (0, b, 0)),
        ],
        out_specs=pl.BlockSpec((br, 4), lambda b: (b, 0)),
        out_shape=jax.ShapeDtypeStruct((n_pad, 4), fdt),
    )(x4, acc)
    x_tilde = x4t[:n_nodes, :3]

    return (x_tilde, m_ij)
